# Initial kernel scaffold; baseline (speedup 1.0000x reference)
import jax
import jax.numpy as jnp
from jax.experimental import pallas as pl


def kernel(x, edge_index, W1, b1, W2, b2):
    raise NotImplementedError("write your pallas kernel here")


# speedup vs baseline: 1.6299x; 1.6299x over previous
import jax
import jax.numpy as jnp
from jax.experimental import pallas as pl


def _combine(dis_ref, s2_ref, g2_ref, o_ref):
    o_ref[...] = dis_ref[...] * (s2_ref[...] + g2_ref[...])


def kernel(x, edge_index, W1, b1, W2, b2):
    src = edge_index[0].astype(jnp.int32)
    dst = edge_index[1].astype(jnp.int32)
    n = x.shape[0]
    deg = jnp.zeros((n,), jnp.float32).at[dst].add(1.0) + 1.0
    dis = jax.lax.rsqrt(deg)
    xs = x * dis[:, None]
    A1 = jnp.zeros((n, x.shape[1]), jnp.float32).at[dst].add(xs[src])
    u = dis[:, None] * A1 + (dis * dis)[:, None] * x
    z = jax.nn.relu(u @ W1 + b1)
    g2 = dis * (z @ W2)[:, 0]
    S2 = jnp.zeros((n,), jnp.float32).at[dst].add(g2[src])
    npad = 100096
    dis_p = jnp.pad(dis, (0, npad - n)).reshape(782, 128)
    s2_p = jnp.pad(S2, (0, npad - n)).reshape(782, 128)
    g2_p = jnp.pad(g2, (0, npad - n)).reshape(782, 128)
    out = pl.pallas_call(
        _combine,
        out_shape=jax.ShapeDtypeStruct((782, 128), jnp.float32),
    )(dis_p, s2_p, g2_p)
    return out.reshape(-1)[:n, None] + b2[None, :]


# speedup vs baseline: 151.3014x; 92.8259x over previous
import functools

import jax
import jax.numpy as jnp
from jax import lax
from jax.experimental import pallas as pl
from jax.experimental.pallas import tpu as pltpu
from jax.experimental.pallas import tpu_sc as plsc

N = 100000
NP = 102400
E = 6400000
NW = 32
K = 16
LW = 128
PD = 1568
G = PD // K
DESC = NW * PD
EPAD = DESC * LW
RS = NP // 16

_mesh = plsc.VectorSubcoreMesh(core_axis_name="c", subcore_axis_name="s")


def _worker(c, s):
    return c * 16 + s


def _deg_body(dst_hbm, zeros_hbm, out_hbm, acc_sp, dstv, ones_v, ssem):
    c = lax.axis_index("c")
    s = lax.axis_index("s")
    pltpu.sync_copy(zeros_hbm.at[pl.ds(s * RS, RS)], acc_sp.at[pl.ds(s * RS, RS)])
    for i in range(LW // 16):
        ones_v[pl.ds(i * 16, 16)] = jnp.ones((16,), jnp.float32)
    plsc.subcore_barrier()
    d0 = _worker(c, s) * PD

    def body(g, carry):
        d = d0 + g * K
        pltpu.sync_copy(dst_hbm.at[pl.ds(d, K)], dstv)
        cps = [pltpu.async_copy(ones_v, acc_sp.at[dstv.at[j]], ssem, add=True)
               for j in range(K)]
        for cp in cps:
            cp.wait()
        return carry

    lax.fori_loop(0, G, body, 0)
    plsc.subcore_barrier()
    pltpu.sync_copy(acc_sp.at[pl.ds(s * RS, RS)], out_hbm.at[c, pl.ds(s * RS, RS)])


_sc_deg = functools.partial(
    pl.kernel,
    out_type=jax.ShapeDtypeStruct((2, NP), jnp.float32),
    mesh=_mesh,
    compiler_params=pltpu.CompilerParams(use_tc_tiling_on_sc=False),
    scratch_types=[
        pltpu.VMEM_SHARED((NP,), jnp.float32),
        pltpu.VMEM((K, LW), jnp.int32),
        pltpu.VMEM((LW,), jnp.float32),
        pltpu.SemaphoreType.DMA,
    ],
)(_deg_body)


def _agg4_body(xs_hbm, src_hbm, dst_hbm, zeros_hbm, out_hbm,
               xs_sp, acc_sp, srcv, dstv, rows, gsem, ssem):
    c = lax.axis_index("c")
    s = lax.axis_index("s")
    pltpu.sync_copy(xs_hbm.at[pl.ds(s * RS, RS)], xs_sp.at[pl.ds(s * RS, RS)])
    pltpu.sync_copy(zeros_hbm.at[pl.ds(s * RS, RS)], acc_sp.at[pl.ds(s * RS, RS)])
    plsc.subcore_barrier()
    d0 = _worker(c, s) * PD

    def body(g, carry):
        d = d0 + g * K
        pltpu.sync_copy(src_hbm.at[pl.ds(d, K)], srcv)
        pltpu.sync_copy(dst_hbm.at[pl.ds(d, K)], dstv)
        gcs = [pltpu.async_copy(xs_sp.at[srcv.at[j]], rows.at[j], gsem)
               for j in range(K)]
        for cp in gcs:
            cp.wait()
        scs = [pltpu.async_copy(rows.at[j], acc_sp.at[dstv.at[j]], ssem, add=True)
               for j in range(K)]
        for cp in scs:
            cp.wait()
        return carry

    lax.fori_loop(0, G, body, 0)
    plsc.subcore_barrier()
    pltpu.sync_copy(acc_sp.at[pl.ds(s * RS, RS)], out_hbm.at[c, pl.ds(s * RS, RS)])


_sc_agg4 = functools.partial(
    pl.kernel,
    out_type=jax.ShapeDtypeStruct((2, NP, 8), jnp.float32),
    mesh=_mesh,
    compiler_params=pltpu.CompilerParams(use_tc_tiling_on_sc=False),
    scratch_types=[
        pltpu.VMEM_SHARED((NP, 8), jnp.float32),
        pltpu.VMEM_SHARED((NP, 8), jnp.float32),
        pltpu.VMEM((K, LW), jnp.int32),
        pltpu.VMEM((K, LW), jnp.int32),
        pltpu.VMEM((K, LW, 8), jnp.float32),
        pltpu.SemaphoreType.DMA,
        pltpu.SemaphoreType.DMA,
    ],
)(_agg4_body)


def _agg1_body(g2_hbm, src_hbm, dst_hbm, zeros_hbm, out_hbm,
               g2_sp, acc_sp, srcv, dstv, rows, gsem, ssem):
    c = lax.axis_index("c")
    s = lax.axis_index("s")
    pltpu.sync_copy(g2_hbm.at[pl.ds(s * RS, RS)], g2_sp.at[pl.ds(s * RS, RS)])
    pltpu.sync_copy(zeros_hbm.at[pl.ds(s * RS, RS)], acc_sp.at[pl.ds(s * RS, RS)])
    plsc.subcore_barrier()
    d0 = _worker(c, s) * PD

    def body(g, carry):
        d = d0 + g * K
        pltpu.sync_copy(src_hbm.at[pl.ds(d, K)], srcv)
        pltpu.sync_copy(dst_hbm.at[pl.ds(d, K)], dstv)
        gcs = [pltpu.async_copy(g2_sp.at[srcv.at[j]], rows.at[j], gsem)
               for j in range(K)]
        for cp in gcs:
            cp.wait()
        scs = [pltpu.async_copy(rows.at[j], acc_sp.at[dstv.at[j]], ssem, add=True)
               for j in range(K)]
        for cp in scs:
            cp.wait()
        return carry

    lax.fori_loop(0, G, body, 0)
    plsc.subcore_barrier()
    pltpu.sync_copy(acc_sp.at[pl.ds(s * RS, RS)], out_hbm.at[c, pl.ds(s * RS, RS)])


_sc_agg1 = functools.partial(
    pl.kernel,
    out_type=jax.ShapeDtypeStruct((2, NP), jnp.float32),
    mesh=_mesh,
    compiler_params=pltpu.CompilerParams(use_tc_tiling_on_sc=False),
    scratch_types=[
        pltpu.VMEM_SHARED((NP,), jnp.float32),
        pltpu.VMEM_SHARED((NP,), jnp.float32),
        pltpu.VMEM((K, LW), jnp.int32),
        pltpu.VMEM((K, LW), jnp.int32),
        pltpu.VMEM((K, LW), jnp.float32),
        pltpu.SemaphoreType.DMA,
        pltpu.SemaphoreType.DMA,
    ],
)(_agg1_body)


_BL = 512
_GRID = NP // _BL


def _dis(degp):
    return lax.rsqrt(degp[0:1, :] + degp[1:2, :] + 1.0)


def _tc1_body(degp_ref, xt_ref, xst_ref):
    xst_ref[0:4, :] = xt_ref[...] * _dis(degp_ref[...])
    xst_ref[4:8, :] = jnp.zeros((4, _BL), jnp.float32)


def _tc2_body(degp_ref, a1t_ref, xt_ref, w1t_ref, b1_ref, w2t_ref, g2_ref):
    dis = _dis(degp_ref[...])
    a1 = a1t_ref[0, 0:4] + a1t_ref[1, 0:4]
    u = dis * a1 + (dis * dis) * xt_ref[...]
    h = jnp.dot(w1t_ref[...], u) + b1_ref[...]
    z = jnp.maximum(h, 0.0)
    h2 = jnp.dot(w2t_ref[...], z)
    col = lax.broadcasted_iota(jnp.int32, (1, _BL), 1) + pl.program_id(0) * _BL
    g2_ref[...] = jnp.where(col < N, dis * h2, 0.0)


def _tc3_body(degp_ref, s2p_ref, g2_ref, b2_ref, out_ref):
    dis = _dis(degp_ref[...])
    s2 = s2p_ref[0:1, :] + s2p_ref[1:2, :]
    out_ref[...] = dis * (s2 + g2_ref[...]) + b2_ref[0, 0]


def _tc1_call(degp, x_t):
    return pl.pallas_call(
        _tc1_body,
        grid=(_GRID,),
        in_specs=[
            pl.BlockSpec((2, _BL), lambda b: (0, b)),
            pl.BlockSpec((4, _BL), lambda b: (0, b)),
        ],
        out_specs=pl.BlockSpec((8, _BL), lambda b: (0, b)),
        out_shape=jax.ShapeDtypeStruct((8, NP), jnp.float32),
    )(degp, x_t)


def _tc2_call(degp, a1t, x_t, W1, b1, W2):
    return pl.pallas_call(
        _tc2_body,
        grid=(_GRID,),
        in_specs=[
            pl.BlockSpec((2, _BL), lambda b: (0, b)),
            pl.BlockSpec((2, 8, _BL), lambda b: (0, 0, b)),
            pl.BlockSpec((4, _BL), lambda b: (0, b)),
            pl.BlockSpec((16, 4), lambda b: (0, 0)),
            pl.BlockSpec((16, 1), lambda b: (0, 0)),
            pl.BlockSpec((1, 16), lambda b: (0, 0)),
        ],
        out_specs=pl.BlockSpec((1, _BL), lambda b: (0, b)),
        out_shape=jax.ShapeDtypeStruct((1, NP), jnp.float32),
    )(degp, a1t, x_t, W1.T, b1.reshape(16, 1), W2.T)


def _tc3_call(degp, s2p, g2, b2):
    return pl.pallas_call(
        _tc3_body,
        grid=(_GRID,),
        in_specs=[
            pl.BlockSpec((2, _BL), lambda b: (0, b)),
            pl.BlockSpec((2, _BL), lambda b: (0, b)),
            pl.BlockSpec((1, _BL), lambda b: (0, b)),
            pl.BlockSpec((1, 1), lambda b: (0, 0)),
        ],
        out_specs=pl.BlockSpec((1, _BL), lambda b: (0, b)),
        out_shape=jax.ShapeDtypeStruct((1, NP), jnp.float32),
    )(degp, s2p, g2, b2.reshape(1, 1))


def kernel(x, edge_index, W1, b1, W2, b2):
    src = edge_index[0].astype(jnp.int32)
    dst = edge_index[1].astype(jnp.int32)
    pad = EPAD - E
    pad_idx = (N + jnp.arange(pad, dtype=jnp.int32) % (NP - N))
    src_p = jnp.concatenate([src, pad_idx]).reshape(DESC, LW)
    dst_p = jnp.concatenate([dst, pad_idx]).reshape(DESC, LW)

    x_t = jnp.pad(x.T, ((0, 0), (0, NP - N)))
    zeros_n = jnp.zeros((NP,), jnp.float32)
    zeros_n8 = jnp.zeros((NP, 8), jnp.float32)

    degp = _sc_deg(dst_p, zeros_n)
    xst = _tc1_call(degp, x_t)
    xs = xst.T.reshape(NP, 8)
    a1p = _sc_agg4(xs, src_p, dst_p, zeros_n8)
    a1t = a1p.transpose(0, 2, 1)
    g2 = _tc2_call(degp, a1t, x_t, W1, b1, W2)
    s2p = _sc_agg1(g2.reshape(NP), src_p, dst_p, zeros_n)
    out = _tc3_call(degp, s2p, g2, b2)
    return out[0, :N].reshape(N, 1)


# speedup vs baseline: 167.4652x; 1.1068x over previous
import functools

import jax
import jax.numpy as jnp
from jax import lax
from jax.experimental import pallas as pl
from jax.experimental.pallas import tpu as pltpu
from jax.experimental.pallas import tpu_sc as plsc

N = 100000
NP = 102400
E = 6400000
NW = 32
K = 16
LW = 128
PD = 1568
G = PD // K
DESC = NW * PD
EPAD = DESC * LW
RS = NP // 16

_mesh = plsc.VectorSubcoreMesh(core_axis_name="c", subcore_axis_name="s")


def _worker(c, s):
    return c * 16 + s


def _deg_body(dst_hbm, zeros_hbm, out_hbm, acc_sp, dstv, ones_v, isem, ssem):
    c = lax.axis_index("c")
    s = lax.axis_index("s")
    pltpu.sync_copy(zeros_hbm.at[pl.ds(s * RS, RS)], acc_sp.at[pl.ds(s * RS, RS)])
    for i in range(LW // 16):
        ones_v[pl.ds(i * 16, 16)] = jnp.ones((16,), jnp.float32)
    plsc.subcore_barrier()
    d0 = _worker(c, s) * PD
    pltpu.async_copy(dst_hbm.at[pl.ds(d0, K)], dstv.at[0], isem.at[0])

    def body(g, carry):
        b = g % 2

        @pl.when(g + 1 < G)
        def _():
            d1 = d0 + (g + 1) * K
            pltpu.async_copy(dst_hbm.at[pl.ds(d1, K)], dstv.at[1 - b],
                             isem.at[1 - b])

        d = d0 + g * K
        pltpu.make_async_copy(dst_hbm.at[pl.ds(d, K)], dstv.at[b],
                              isem.at[b]).wait()
        cps = [pltpu.async_copy(ones_v, acc_sp.at[dstv.at[b, j]], ssem, add=True)
               for j in range(K)]
        for cp in cps:
            cp.wait()
        return carry

    lax.fori_loop(0, G, body, 0)
    plsc.subcore_barrier()
    pltpu.sync_copy(acc_sp.at[pl.ds(s * RS, RS)], out_hbm.at[c, pl.ds(s * RS, RS)])


_sc_deg = functools.partial(
    pl.kernel,
    out_type=jax.ShapeDtypeStruct((2, NP), jnp.float32),
    mesh=_mesh,
    compiler_params=pltpu.CompilerParams(use_tc_tiling_on_sc=False),
    scratch_types=[
        pltpu.VMEM_SHARED((NP,), jnp.float32),
        pltpu.VMEM((2, K, LW), jnp.int32),
        pltpu.VMEM((LW,), jnp.float32),
        pltpu.SemaphoreType.DMA((2,)),
        pltpu.SemaphoreType.DMA,
    ],
)(_deg_body)


def _agg4_body(xs_hbm, src_hbm, dst_hbm, zeros_hbm, out_hbm,
               xs_sp, acc_sp, srcv, dstv, rows, isem, gsem, ssem):
    c = lax.axis_index("c")
    s = lax.axis_index("s")
    pltpu.sync_copy(xs_hbm.at[pl.ds(s * RS, RS)], xs_sp.at[pl.ds(s * RS, RS)])
    pltpu.sync_copy(zeros_hbm.at[pl.ds(s * RS, RS)], acc_sp.at[pl.ds(s * RS, RS)])
    plsc.subcore_barrier()
    d0 = _worker(c, s) * PD
    pltpu.async_copy(src_hbm.at[pl.ds(d0, K)], srcv.at[0], isem.at[0])
    pltpu.async_copy(dst_hbm.at[pl.ds(d0, K)], dstv.at[0], isem.at[0])

    def body(g, carry):
        b = g % 2

        @pl.when(g + 1 < G)
        def _():
            d1 = d0 + (g + 1) * K
            pltpu.async_copy(src_hbm.at[pl.ds(d1, K)], srcv.at[1 - b],
                             isem.at[1 - b])
            pltpu.async_copy(dst_hbm.at[pl.ds(d1, K)], dstv.at[1 - b],
                             isem.at[1 - b])

        d = d0 + g * K
        pltpu.make_async_copy(src_hbm.at[pl.ds(d, K)], srcv.at[b],
                              isem.at[b]).wait()
        pltpu.make_async_copy(dst_hbm.at[pl.ds(d, K)], dstv.at[b],
                              isem.at[b]).wait()
        gcs = [pltpu.async_copy(xs_sp.at[srcv.at[b, j]], rows.at[j], gsem)
               for j in range(K)]
        for cp in gcs:
            cp.wait()
        scs = [pltpu.async_copy(rows.at[j], acc_sp.at[dstv.at[b, j]], ssem,
                                add=True)
               for j in range(K)]
        for cp in scs:
            cp.wait()
        return carry

    lax.fori_loop(0, G, body, 0)
    plsc.subcore_barrier()
    pltpu.sync_copy(acc_sp.at[pl.ds(s * RS, RS)], out_hbm.at[c, pl.ds(s * RS, RS)])


_sc_agg4 = functools.partial(
    pl.kernel,
    out_type=jax.ShapeDtypeStruct((2, NP, 8), jnp.float32),
    mesh=_mesh,
    compiler_params=pltpu.CompilerParams(use_tc_tiling_on_sc=False),
    scratch_types=[
        pltpu.VMEM_SHARED((NP, 8), jnp.float32),
        pltpu.VMEM_SHARED((NP, 8), jnp.float32),
        pltpu.VMEM((2, K, LW), jnp.int32),
        pltpu.VMEM((2, K, LW), jnp.int32),
        pltpu.VMEM((K, LW, 8), jnp.float32),
        pltpu.SemaphoreType.DMA((2,)),
        pltpu.SemaphoreType.DMA,
        pltpu.SemaphoreType.DMA,
    ],
)(_agg4_body)


def _agg1_body(g2_hbm, src_hbm, dst_hbm, zeros_hbm, out_hbm,
               g2_sp, acc_sp, srcv, dstv, rows, isem, gsem, ssem):
    c = lax.axis_index("c")
    s = lax.axis_index("s")
    pltpu.sync_copy(g2_hbm.at[pl.ds(s * RS, RS)], g2_sp.at[pl.ds(s * RS, RS)])
    pltpu.sync_copy(zeros_hbm.at[pl.ds(s * RS, RS)], acc_sp.at[pl.ds(s * RS, RS)])
    plsc.subcore_barrier()
    d0 = _worker(c, s) * PD
    pltpu.async_copy(src_hbm.at[pl.ds(d0, K)], srcv.at[0], isem.at[0])
    pltpu.async_copy(dst_hbm.at[pl.ds(d0, K)], dstv.at[0], isem.at[0])

    def body(g, carry):
        b = g % 2

        @pl.when(g + 1 < G)
        def _():
            d1 = d0 + (g + 1) * K
            pltpu.async_copy(src_hbm.at[pl.ds(d1, K)], srcv.at[1 - b],
                             isem.at[1 - b])
            pltpu.async_copy(dst_hbm.at[pl.ds(d1, K)], dstv.at[1 - b],
                             isem.at[1 - b])

        d = d0 + g * K
        pltpu.make_async_copy(src_hbm.at[pl.ds(d, K)], srcv.at[b],
                              isem.at[b]).wait()
        pltpu.make_async_copy(dst_hbm.at[pl.ds(d, K)], dstv.at[b],
                              isem.at[b]).wait()
        gcs = [pltpu.async_copy(g2_sp.at[srcv.at[b, j]], rows.at[j], gsem)
               for j in range(K)]
        for cp in gcs:
            cp.wait()
        scs = [pltpu.async_copy(rows.at[j], acc_sp.at[dstv.at[b, j]], ssem,
                                add=True)
               for j in range(K)]
        for cp in scs:
            cp.wait()
        return carry

    lax.fori_loop(0, G, body, 0)
    plsc.subcore_barrier()
    pltpu.sync_copy(acc_sp.at[pl.ds(s * RS, RS)], out_hbm.at[c, pl.ds(s * RS, RS)])


_sc_agg1 = functools.partial(
    pl.kernel,
    out_type=jax.ShapeDtypeStruct((2, NP), jnp.float32),
    mesh=_mesh,
    compiler_params=pltpu.CompilerParams(use_tc_tiling_on_sc=False),
    scratch_types=[
        pltpu.VMEM_SHARED((NP,), jnp.float32),
        pltpu.VMEM_SHARED((NP,), jnp.float32),
        pltpu.VMEM((2, K, LW), jnp.int32),
        pltpu.VMEM((2, K, LW), jnp.int32),
        pltpu.VMEM((K, LW), jnp.float32),
        pltpu.SemaphoreType.DMA((2,)),
        pltpu.SemaphoreType.DMA,
        pltpu.SemaphoreType.DMA,
    ],
)(_agg1_body)


_BL = 512
_GRID = NP // _BL


def _dis(degp):
    return lax.rsqrt(degp[0:1, :] + degp[1:2, :] + 1.0)


def _tc1_body(degp_ref, xt_ref, xst_ref):
    xst_ref[0:4, :] = xt_ref[...] * _dis(degp_ref[...])
    xst_ref[4:8, :] = jnp.zeros((4, _BL), jnp.float32)


def _tc2_body(degp_ref, a1t_ref, xt_ref, w1t_ref, b1_ref, w2t_ref, g2_ref):
    dis = _dis(degp_ref[...])
    a1 = a1t_ref[0, 0:4] + a1t_ref[1, 0:4]
    u = dis * a1 + (dis * dis) * xt_ref[...]
    h = jnp.dot(w1t_ref[...], u) + b1_ref[...]
    z = jnp.maximum(h, 0.0)
    h2 = jnp.dot(w2t_ref[...], z)
    col = lax.broadcasted_iota(jnp.int32, (1, _BL), 1) + pl.program_id(0) * _BL
    g2_ref[...] = jnp.where(col < N, dis * h2, 0.0)


def _tc3_body(degp_ref, s2p_ref, g2_ref, b2_ref, out_ref):
    dis = _dis(degp_ref[...])
    s2 = s2p_ref[0:1, :] + s2p_ref[1:2, :]
    out_ref[...] = dis * (s2 + g2_ref[...]) + b2_ref[0, 0]


def _tc1_call(degp, x_t):
    return pl.pallas_call(
        _tc1_body,
        grid=(_GRID,),
        in_specs=[
            pl.BlockSpec((2, _BL), lambda b: (0, b)),
            pl.BlockSpec((4, _BL), lambda b: (0, b)),
        ],
        out_specs=pl.BlockSpec((8, _BL), lambda b: (0, b)),
        out_shape=jax.ShapeDtypeStruct((8, NP), jnp.float32),
    )(degp, x_t)


def _tc2_call(degp, a1t, x_t, W1, b1, W2):
    return pl.pallas_call(
        _tc2_body,
        grid=(_GRID,),
        in_specs=[
            pl.BlockSpec((2, _BL), lambda b: (0, b)),
            pl.BlockSpec((2, 8, _BL), lambda b: (0, 0, b)),
            pl.BlockSpec((4, _BL), lambda b: (0, b)),
            pl.BlockSpec((16, 4), lambda b: (0, 0)),
            pl.BlockSpec((16, 1), lambda b: (0, 0)),
            pl.BlockSpec((1, 16), lambda b: (0, 0)),
        ],
        out_specs=pl.BlockSpec((1, _BL), lambda b: (0, b)),
        out_shape=jax.ShapeDtypeStruct((1, NP), jnp.float32),
    )(degp, a1t, x_t, W1.T, b1.reshape(16, 1), W2.T)


def _tc3_call(degp, s2p, g2, b2):
    return pl.pallas_call(
        _tc3_body,
        grid=(_GRID,),
        in_specs=[
            pl.BlockSpec((2, _BL), lambda b: (0, b)),
            pl.BlockSpec((2, _BL), lambda b: (0, b)),
            pl.BlockSpec((1, _BL), lambda b: (0, b)),
            pl.BlockSpec((1, 1), lambda b: (0, 0)),
        ],
        out_specs=pl.BlockSpec((1, _BL), lambda b: (0, b)),
        out_shape=jax.ShapeDtypeStruct((1, NP), jnp.float32),
    )(degp, s2p, g2, b2.reshape(1, 1))


def kernel(x, edge_index, W1, b1, W2, b2):
    src = edge_index[0].astype(jnp.int32)
    dst = edge_index[1].astype(jnp.int32)
    pad = EPAD - E
    pad_idx = (N + jnp.arange(pad, dtype=jnp.int32) % (NP - N))
    src_p = jnp.concatenate([src, pad_idx]).reshape(DESC, LW)
    dst_p = jnp.concatenate([dst, pad_idx]).reshape(DESC, LW)

    x_t = jnp.pad(x.T, ((0, 0), (0, NP - N)))
    zeros_n = jnp.zeros((NP,), jnp.float32)
    zeros_n8 = jnp.zeros((NP, 8), jnp.float32)

    degp = _sc_deg(dst_p, zeros_n)
    xst = _tc1_call(degp, x_t)
    xs = xst.T.reshape(NP, 8)
    a1p = _sc_agg4(xs, src_p, dst_p, zeros_n8)
    a1t = a1p.transpose(0, 2, 1)
    g2 = _tc2_call(degp, a1t, x_t, W1, b1, W2)
    s2p = _sc_agg1(g2.reshape(NP), src_p, dst_p, zeros_n)
    out = _tc3_call(degp, s2p, g2, b2)
    return out[0, :N].reshape(N, 1)


# speedup vs baseline: 209.0812x; 1.2485x over previous
import functools

import jax
import jax.numpy as jnp
from jax import lax
from jax.experimental import pallas as pl
from jax.experimental.pallas import tpu as pltpu
from jax.experimental.pallas import tpu_sc as plsc

N = 100000
NP = 102400
E = 6400000
NW = 32
K = 16
LW = 128
PD = 1568
G = PD // K
DESC = NW * PD
EPAD = DESC * LW
RS = NP // 16
KB = 4
LWB = 512
PDB = (PD * LW) // LWB
DESCB = EPAD // LWB

_mesh = plsc.VectorSubcoreMesh(core_axis_name="c", subcore_axis_name="s")


def _worker(c, s):
    return c * 16 + s


def _deg_body(dst_hbm, zeros_hbm, out_hbm, acc_sp, dstv, ones_v, isem, ssem):
    c = lax.axis_index("c")
    s = lax.axis_index("s")
    pltpu.sync_copy(zeros_hbm.at[pl.ds(s * RS, RS)], acc_sp.at[pl.ds(s * RS, RS)])
    for i in range(LW // 16):
        ones_v[pl.ds(i * 16, 16)] = jnp.ones((16,), jnp.float32)
    plsc.subcore_barrier()
    d0 = _worker(c, s) * PD
    pltpu.async_copy(dst_hbm.at[pl.ds(d0, K)], dstv.at[0], isem.at[0])

    def body(g, carry):
        b = g % 2

        @pl.when(g + 1 < G)
        def _():
            d1 = d0 + (g + 1) * K
            pltpu.async_copy(dst_hbm.at[pl.ds(d1, K)], dstv.at[1 - b],
                             isem.at[1 - b])

        d = d0 + g * K
        pltpu.make_async_copy(dst_hbm.at[pl.ds(d, K)], dstv.at[b],
                              isem.at[b]).wait()
        cps = [pltpu.async_copy(ones_v, acc_sp.at[dstv.at[b, j]], ssem, add=True)
               for j in range(K)]
        for cp in cps:
            cp.wait()
        return carry

    lax.fori_loop(0, G, body, 0)
    plsc.subcore_barrier()
    pltpu.sync_copy(acc_sp.at[pl.ds(s * RS, RS)], out_hbm.at[c, pl.ds(s * RS, RS)])


_sc_deg = functools.partial(
    pl.kernel,
    out_type=jax.ShapeDtypeStruct((2, NP), jnp.float32),
    mesh=_mesh,
    compiler_params=pltpu.CompilerParams(use_tc_tiling_on_sc=False),
    scratch_types=[
        pltpu.VMEM_SHARED((NP,), jnp.float32),
        pltpu.VMEM((2, K, LW), jnp.int32),
        pltpu.VMEM((LW,), jnp.float32),
        pltpu.SemaphoreType.DMA((2,)),
        pltpu.SemaphoreType.DMA,
    ],
)(_deg_body)


def _agg4_body(xs_hbm, src_hbm, dst_hbm, zeros_hbm, out_hbm,
               xs_sp, acc_sp, srcv, dstv, rows, isem, gsem, ssem):
    c = lax.axis_index("c")
    s = lax.axis_index("s")
    pltpu.sync_copy(xs_hbm.at[pl.ds(s * RS, RS)], xs_sp.at[pl.ds(s * RS, RS)])
    pltpu.sync_copy(zeros_hbm.at[pl.ds(s * RS, RS)], acc_sp.at[pl.ds(s * RS, RS)])
    plsc.subcore_barrier()
    d0 = _worker(c, s) * PDB
    pltpu.async_copy(src_hbm.at[pl.ds(d0, KB)], srcv.at[0], isem.at[0])
    pltpu.async_copy(dst_hbm.at[pl.ds(d0, KB)], dstv.at[0], isem.at[0])

    def body(g, carry):
        b = g % 2

        @pl.when(g + 1 < G)
        def _():
            d1 = d0 + (g + 1) * KB
            pltpu.async_copy(src_hbm.at[pl.ds(d1, KB)], srcv.at[1 - b],
                             isem.at[1 - b])
            pltpu.async_copy(dst_hbm.at[pl.ds(d1, KB)], dstv.at[1 - b],
                             isem.at[1 - b])

        d = d0 + g * KB
        pltpu.make_async_copy(src_hbm.at[pl.ds(d, KB)], srcv.at[b],
                              isem.at[b]).wait()
        pltpu.make_async_copy(dst_hbm.at[pl.ds(d, KB)], dstv.at[b],
                              isem.at[b]).wait()
        gcs = [pltpu.async_copy(xs_sp.at[srcv.at[b, j]], rows.at[j],
                                gsem.at[j])
               for j in range(KB)]
        scs = []
        for j in range(KB):
            gcs[j].wait()
            scs.append(pltpu.async_copy(rows.at[j], acc_sp.at[dstv.at[b, j]],
                                        ssem, add=True))
        for cp in scs:
            cp.wait()
        return carry

    lax.fori_loop(0, G, body, 0)
    plsc.subcore_barrier()
    pltpu.sync_copy(acc_sp.at[pl.ds(s * RS, RS)], out_hbm.at[c, pl.ds(s * RS, RS)])


_sc_agg4 = functools.partial(
    pl.kernel,
    out_type=jax.ShapeDtypeStruct((2, NP, 8), jnp.float32),
    mesh=_mesh,
    compiler_params=pltpu.CompilerParams(use_tc_tiling_on_sc=False),
    scratch_types=[
        pltpu.VMEM_SHARED((NP, 8), jnp.float32),
        pltpu.VMEM_SHARED((NP, 8), jnp.float32),
        pltpu.VMEM((2, KB, LWB), jnp.int32),
        pltpu.VMEM((2, KB, LWB), jnp.int32),
        pltpu.VMEM((KB, LWB, 8), jnp.float32),
        pltpu.SemaphoreType.DMA((2,)),
        pltpu.SemaphoreType.DMA((KB,)),
        pltpu.SemaphoreType.DMA,
    ],
)(_agg4_body)


def _agg1_body(g2_hbm, src_hbm, dst_hbm, zeros_hbm, out_hbm,
               g2_sp, acc_sp, srcv, dstv, rows, isem, gsem, ssem):
    c = lax.axis_index("c")
    s = lax.axis_index("s")
    pltpu.sync_copy(g2_hbm.at[pl.ds(s * RS, RS)], g2_sp.at[pl.ds(s * RS, RS)])
    pltpu.sync_copy(zeros_hbm.at[pl.ds(s * RS, RS)], acc_sp.at[pl.ds(s * RS, RS)])
    plsc.subcore_barrier()
    d0 = _worker(c, s) * PD
    pltpu.async_copy(src_hbm.at[pl.ds(d0, K)], srcv.at[0], isem.at[0])
    pltpu.async_copy(dst_hbm.at[pl.ds(d0, K)], dstv.at[0], isem.at[0])

    def body(g, carry):
        b = g % 2

        @pl.when(g + 1 < G)
        def _():
            d1 = d0 + (g + 1) * K
            pltpu.async_copy(src_hbm.at[pl.ds(d1, K)], srcv.at[1 - b],
                             isem.at[1 - b])
            pltpu.async_copy(dst_hbm.at[pl.ds(d1, K)], dstv.at[1 - b],
                             isem.at[1 - b])

        d = d0 + g * K
        pltpu.make_async_copy(src_hbm.at[pl.ds(d, K)], srcv.at[b],
                              isem.at[b]).wait()
        pltpu.make_async_copy(dst_hbm.at[pl.ds(d, K)], dstv.at[b],
                              isem.at[b]).wait()
        gcs = [pltpu.async_copy(g2_sp.at[srcv.at[b, j]], rows.at[j],
                                gsem.at[j])
               for j in range(K)]
        scs = []
        for j in range(K):
            gcs[j].wait()
            scs.append(pltpu.async_copy(rows.at[j], acc_sp.at[dstv.at[b, j]],
                                        ssem, add=True))
        for cp in scs:
            cp.wait()
        return carry

    lax.fori_loop(0, G, body, 0)
    plsc.subcore_barrier()
    pltpu.sync_copy(acc_sp.at[pl.ds(s * RS, RS)], out_hbm.at[c, pl.ds(s * RS, RS)])


_sc_agg1 = functools.partial(
    pl.kernel,
    out_type=jax.ShapeDtypeStruct((2, NP), jnp.float32),
    mesh=_mesh,
    compiler_params=pltpu.CompilerParams(use_tc_tiling_on_sc=False),
    scratch_types=[
        pltpu.VMEM_SHARED((NP,), jnp.float32),
        pltpu.VMEM_SHARED((NP,), jnp.float32),
        pltpu.VMEM((2, K, LW), jnp.int32),
        pltpu.VMEM((2, K, LW), jnp.int32),
        pltpu.VMEM((K, LW), jnp.float32),
        pltpu.SemaphoreType.DMA((2,)),
        pltpu.SemaphoreType.DMA((K,)),
        pltpu.SemaphoreType.DMA,
    ],
)(_agg1_body)


_BL = 512
_GRID = NP // _BL


def _dis(degp):
    return lax.rsqrt(degp[0:1, :] + degp[1:2, :] + 1.0)


def _tc1_body(degp_ref, xt_ref, xst_ref):
    xst_ref[0:4, :] = xt_ref[...] * _dis(degp_ref[...])
    xst_ref[4:8, :] = jnp.zeros((4, _BL), jnp.float32)


def _tc2_body(degp_ref, a1t_ref, xt_ref, w1t_ref, b1_ref, w2t_ref, g2_ref):
    dis = _dis(degp_ref[...])
    a1 = a1t_ref[0, 0:4] + a1t_ref[1, 0:4]
    u = dis * a1 + (dis * dis) * xt_ref[...]
    h = jnp.dot(w1t_ref[...], u) + b1_ref[...]
    z = jnp.maximum(h, 0.0)
    h2 = jnp.dot(w2t_ref[...], z)
    col = lax.broadcasted_iota(jnp.int32, (1, _BL), 1) + pl.program_id(0) * _BL
    g2_ref[...] = jnp.where(col < N, dis * h2, 0.0)


def _tc3_body(degp_ref, s2p_ref, g2_ref, b2_ref, out_ref):
    dis = _dis(degp_ref[...])
    s2 = s2p_ref[0:1, :] + s2p_ref[1:2, :]
    out_ref[...] = dis * (s2 + g2_ref[...]) + b2_ref[0, 0]


def _tc1_call(degp, x_t):
    return pl.pallas_call(
        _tc1_body,
        grid=(_GRID,),
        in_specs=[
            pl.BlockSpec((2, _BL), lambda b: (0, b)),
            pl.BlockSpec((4, _BL), lambda b: (0, b)),
        ],
        out_specs=pl.BlockSpec((8, _BL), lambda b: (0, b)),
        out_shape=jax.ShapeDtypeStruct((8, NP), jnp.float32),
    )(degp, x_t)


def _tc2_call(degp, a1t, x_t, W1, b1, W2):
    return pl.pallas_call(
        _tc2_body,
        grid=(_GRID,),
        in_specs=[
            pl.BlockSpec((2, _BL), lambda b: (0, b)),
            pl.BlockSpec((2, 8, _BL), lambda b: (0, 0, b)),
            pl.BlockSpec((4, _BL), lambda b: (0, b)),
            pl.BlockSpec((16, 4), lambda b: (0, 0)),
            pl.BlockSpec((16, 1), lambda b: (0, 0)),
            pl.BlockSpec((1, 16), lambda b: (0, 0)),
        ],
        out_specs=pl.BlockSpec((1, _BL), lambda b: (0, b)),
        out_shape=jax.ShapeDtypeStruct((1, NP), jnp.float32),
    )(degp, a1t, x_t, W1.T, b1.reshape(16, 1), W2.T)


def _tc3_call(degp, s2p, g2, b2):
    return pl.pallas_call(
        _tc3_body,
        grid=(_GRID,),
        in_specs=[
            pl.BlockSpec((2, _BL), lambda b: (0, b)),
            pl.BlockSpec((2, _BL), lambda b: (0, b)),
            pl.BlockSpec((1, _BL), lambda b: (0, b)),
            pl.BlockSpec((1, 1), lambda b: (0, 0)),
        ],
        out_specs=pl.BlockSpec((1, _BL), lambda b: (0, b)),
        out_shape=jax.ShapeDtypeStruct((1, NP), jnp.float32),
    )(degp, s2p, g2, b2.reshape(1, 1))


def kernel(x, edge_index, W1, b1, W2, b2):
    src = edge_index[0].astype(jnp.int32)
    dst = edge_index[1].astype(jnp.int32)
    pad = EPAD - E
    pad_idx = (N + jnp.arange(pad, dtype=jnp.int32) % (NP - N))
    src_flat = jnp.concatenate([src, pad_idx])
    dst_flat = jnp.concatenate([dst, pad_idx])
    src_p = src_flat.reshape(DESC, LW)
    dst_p = dst_flat.reshape(DESC, LW)
    src_pb = src_flat.reshape(DESCB, LWB)
    dst_pb = dst_flat.reshape(DESCB, LWB)

    x_t = jnp.pad(x.T, ((0, 0), (0, NP - N)))
    zeros_n = jnp.zeros((NP,), jnp.float32)
    zeros_n8 = jnp.zeros((NP, 8), jnp.float32)

    degp = _sc_deg(dst_p, zeros_n)
    xst = _tc1_call(degp, x_t)
    xs = xst.T.reshape(NP, 8)
    a1p = _sc_agg4(xs, src_pb, dst_pb, zeros_n8)
    a1t = a1p.transpose(0, 2, 1)
    g2 = _tc2_call(degp, a1t, x_t, W1, b1, W2)
    s2p = _sc_agg1(g2.reshape(NP), src_p, dst_p, zeros_n)
    out = _tc3_call(degp, s2p, g2, b2)
    return out[0, :N].reshape(N, 1)


# speedup vs baseline: 217.1055x; 1.0384x over previous
import functools

import jax
import jax.numpy as jnp
from jax import lax
from jax.experimental import pallas as pl
from jax.experimental.pallas import tpu as pltpu
from jax.experimental.pallas import tpu_sc as plsc

N = 100000
NP = 102400
E = 6400000
NW = 32
K = 16
LW = 128
PD = 1568
G = PD // K
GLAST = 87
DESC = E // LW
RS = NP // 16
KB = 4
LWB = 512
PDB = (PD * LW) // LWB
DESCB = E // LWB

_mesh = plsc.VectorSubcoreMesh(core_axis_name="c", subcore_axis_name="s")


def _worker(c, s):
    return c * 16 + s


def _deg_body(dst_hbm, zeros_hbm, out_hbm, acc_sp, dstv, ones_v, isem, ssem):
    c = lax.axis_index("c")
    s = lax.axis_index("s")
    pltpu.sync_copy(zeros_hbm.at[pl.ds(s * RS, RS)], acc_sp.at[pl.ds(s * RS, RS)])
    for i in range(LW // 16):
        ones_v[pl.ds(i * 16, 16)] = jnp.ones((16,), jnp.float32)
    plsc.subcore_barrier()
    w = _worker(c, s)
    gw = jnp.where(w == NW - 1, GLAST, G)
    d0 = w * PD
    pltpu.async_copy(dst_hbm.at[pl.ds(d0, K)], dstv.at[0], isem.at[0])

    def body(g, carry):
        b = g % 2

        @pl.when(g + 1 < gw)
        def _():
            d1 = d0 + (g + 1) * K
            pltpu.async_copy(dst_hbm.at[pl.ds(d1, K)], dstv.at[1 - b],
                             isem.at[1 - b])

        d = d0 + g * K
        pltpu.make_async_copy(dst_hbm.at[pl.ds(d, K)], dstv.at[b],
                              isem.at[b]).wait()
        cps = [pltpu.async_copy(ones_v, acc_sp.at[dstv.at[b, j]], ssem, add=True)
               for j in range(K)]
        for cp in cps:
            cp.wait()
        return carry

    lax.fori_loop(0, gw, body, 0)
    plsc.subcore_barrier()
    pltpu.sync_copy(acc_sp.at[pl.ds(s * RS, RS)], out_hbm.at[c, pl.ds(s * RS, RS)])


_sc_deg = functools.partial(
    pl.kernel,
    out_type=jax.ShapeDtypeStruct((2, NP), jnp.float32),
    mesh=_mesh,
    compiler_params=pltpu.CompilerParams(use_tc_tiling_on_sc=False),
    scratch_types=[
        pltpu.VMEM_SHARED((NP,), jnp.float32),
        pltpu.VMEM((2, K, LW), jnp.int32),
        pltpu.VMEM((LW,), jnp.float32),
        pltpu.SemaphoreType.DMA((2,)),
        pltpu.SemaphoreType.DMA,
    ],
)(_deg_body)


def _agg4_body(xs_hbm, src_hbm, dst_hbm, zeros_hbm, out_hbm,
               xs_sp, acc_sp, srcv, dstv, rows, isem, gsem, ssem):
    c = lax.axis_index("c")
    s = lax.axis_index("s")
    pltpu.sync_copy(xs_hbm.at[pl.ds(s * RS, RS)], xs_sp.at[pl.ds(s * RS, RS)])
    pltpu.sync_copy(zeros_hbm.at[pl.ds(s * RS, RS)], acc_sp.at[pl.ds(s * RS, RS)])
    plsc.subcore_barrier()
    w = _worker(c, s)
    gw = jnp.where(w == NW - 1, GLAST, G)
    d0 = w * PDB
    pltpu.async_copy(src_hbm.at[pl.ds(d0, KB)], srcv.at[0], isem.at[0])
    pltpu.async_copy(dst_hbm.at[pl.ds(d0, KB)], dstv.at[0], isem.at[0])

    def body(g, carry):
        b = g % 2

        @pl.when(g + 1 < gw)
        def _():
            d1 = d0 + (g + 1) * KB
            pltpu.async_copy(src_hbm.at[pl.ds(d1, KB)], srcv.at[1 - b],
                             isem.at[1 - b])
            pltpu.async_copy(dst_hbm.at[pl.ds(d1, KB)], dstv.at[1 - b],
                             isem.at[1 - b])

        d = d0 + g * KB
        pltpu.make_async_copy(src_hbm.at[pl.ds(d, KB)], srcv.at[b],
                              isem.at[b]).wait()
        pltpu.make_async_copy(dst_hbm.at[pl.ds(d, KB)], dstv.at[b],
                              isem.at[b]).wait()
        gcs = [pltpu.async_copy(xs_sp.at[srcv.at[b, j]], rows.at[j],
                                gsem.at[j])
               for j in range(KB)]
        scs = []
        for j in range(KB):
            gcs[j].wait()
            scs.append(pltpu.async_copy(rows.at[j], acc_sp.at[dstv.at[b, j]],
                                        ssem, add=True))
        for cp in scs:
            cp.wait()
        return carry

    lax.fori_loop(0, gw, body, 0)
    plsc.subcore_barrier()
    pltpu.sync_copy(acc_sp.at[pl.ds(s * RS, RS)], out_hbm.at[c, pl.ds(s * RS, RS)])


_sc_agg4 = functools.partial(
    pl.kernel,
    out_type=jax.ShapeDtypeStruct((2, NP, 8), jnp.float32),
    mesh=_mesh,
    compiler_params=pltpu.CompilerParams(use_tc_tiling_on_sc=False),
    scratch_types=[
        pltpu.VMEM_SHARED((NP, 8), jnp.float32),
        pltpu.VMEM_SHARED((NP, 8), jnp.float32),
        pltpu.VMEM((2, KB, LWB), jnp.int32),
        pltpu.VMEM((2, KB, LWB), jnp.int32),
        pltpu.VMEM((KB, LWB, 8), jnp.float32),
        pltpu.SemaphoreType.DMA((2,)),
        pltpu.SemaphoreType.DMA((KB,)),
        pltpu.SemaphoreType.DMA,
    ],
)(_agg4_body)


def _agg1_body(g2_hbm, src_hbm, dst_hbm, zeros_hbm, out_hbm,
               g2_sp, acc_sp, srcv, dstv, rows, isem, gsem, ssem):
    c = lax.axis_index("c")
    s = lax.axis_index("s")
    pltpu.sync_copy(g2_hbm.at[pl.ds(s * RS, RS)], g2_sp.at[pl.ds(s * RS, RS)])
    pltpu.sync_copy(zeros_hbm.at[pl.ds(s * RS, RS)], acc_sp.at[pl.ds(s * RS, RS)])
    plsc.subcore_barrier()
    w = _worker(c, s)
    gw = jnp.where(w == NW - 1, GLAST, G)
    d0 = w * PD
    pltpu.async_copy(src_hbm.at[pl.ds(d0, K)], srcv.at[0], isem.at[0])
    pltpu.async_copy(dst_hbm.at[pl.ds(d0, K)], dstv.at[0], isem.at[0])

    def body(g, carry):
        b = g % 2

        @pl.when(g + 1 < gw)
        def _():
            d1 = d0 + (g + 1) * K
            pltpu.async_copy(src_hbm.at[pl.ds(d1, K)], srcv.at[1 - b],
                             isem.at[1 - b])
            pltpu.async_copy(dst_hbm.at[pl.ds(d1, K)], dstv.at[1 - b],
                             isem.at[1 - b])

        d = d0 + g * K
        pltpu.make_async_copy(src_hbm.at[pl.ds(d, K)], srcv.at[b],
                              isem.at[b]).wait()
        pltpu.make_async_copy(dst_hbm.at[pl.ds(d, K)], dstv.at[b],
                              isem.at[b]).wait()
        gcs = [pltpu.async_copy(g2_sp.at[srcv.at[b, j]], rows.at[j],
                                gsem.at[j])
               for j in range(K)]
        scs = []
        for j in range(K):
            gcs[j].wait()
            scs.append(pltpu.async_copy(rows.at[j], acc_sp.at[dstv.at[b, j]],
                                        ssem, add=True))
        for cp in scs:
            cp.wait()
        return carry

    lax.fori_loop(0, gw, body, 0)
    plsc.subcore_barrier()
    pltpu.sync_copy(acc_sp.at[pl.ds(s * RS, RS)], out_hbm.at[c, pl.ds(s * RS, RS)])


_sc_agg1 = functools.partial(
    pl.kernel,
    out_type=jax.ShapeDtypeStruct((2, NP), jnp.float32),
    mesh=_mesh,
    compiler_params=pltpu.CompilerParams(use_tc_tiling_on_sc=False),
    scratch_types=[
        pltpu.VMEM_SHARED((NP,), jnp.float32),
        pltpu.VMEM_SHARED((NP,), jnp.float32),
        pltpu.VMEM((2, K, LW), jnp.int32),
        pltpu.VMEM((2, K, LW), jnp.int32),
        pltpu.VMEM((K, LW), jnp.float32),
        pltpu.SemaphoreType.DMA((2,)),
        pltpu.SemaphoreType.DMA((K,)),
        pltpu.SemaphoreType.DMA,
    ],
)(_agg1_body)


_BL = 512
_GRID = NP // _BL


def _dis(degp):
    return lax.rsqrt(degp[0:1, :] + degp[1:2, :] + 1.0)


def _tc1_body(degp_ref, xt_ref, xst_ref):
    xst_ref[0:4, :] = xt_ref[...] * _dis(degp_ref[...])
    xst_ref[4:8, :] = jnp.zeros((4, _BL), jnp.float32)


def _tc2_body(degp_ref, a1t_ref, xt_ref, w1t_ref, b1_ref, w2t_ref, g2_ref):
    dis = _dis(degp_ref[...])
    a1 = a1t_ref[0, 0:4] + a1t_ref[1, 0:4]
    u = dis * a1 + (dis * dis) * xt_ref[...]
    h = jnp.dot(w1t_ref[...], u) + b1_ref[...]
    z = jnp.maximum(h, 0.0)
    h2 = jnp.dot(w2t_ref[...], z)
    col = lax.broadcasted_iota(jnp.int32, (1, _BL), 1) + pl.program_id(0) * _BL
    g2_ref[...] = jnp.where(col < N, dis * h2, 0.0)


def _tc3_body(degp_ref, s2p_ref, g2_ref, b2_ref, out_ref):
    dis = _dis(degp_ref[...])
    s2 = s2p_ref[0:1, :] + s2p_ref[1:2, :]
    out_ref[...] = dis * (s2 + g2_ref[...]) + b2_ref[0, 0]


def _tc1_call(degp, x_t):
    return pl.pallas_call(
        _tc1_body,
        grid=(_GRID,),
        in_specs=[
            pl.BlockSpec((2, _BL), lambda b: (0, b)),
            pl.BlockSpec((4, _BL), lambda b: (0, b)),
        ],
        out_specs=pl.BlockSpec((8, _BL), lambda b: (0, b)),
        out_shape=jax.ShapeDtypeStruct((8, NP), jnp.float32),
    )(degp, x_t)


def _tc2_call(degp, a1t, x_t, W1, b1, W2):
    return pl.pallas_call(
        _tc2_body,
        grid=(_GRID,),
        in_specs=[
            pl.BlockSpec((2, _BL), lambda b: (0, b)),
            pl.BlockSpec((2, 8, _BL), lambda b: (0, 0, b)),
            pl.BlockSpec((4, _BL), lambda b: (0, b)),
            pl.BlockSpec((16, 4), lambda b: (0, 0)),
            pl.BlockSpec((16, 1), lambda b: (0, 0)),
            pl.BlockSpec((1, 16), lambda b: (0, 0)),
        ],
        out_specs=pl.BlockSpec((1, _BL), lambda b: (0, b)),
        out_shape=jax.ShapeDtypeStruct((1, NP), jnp.float32),
    )(degp, a1t, x_t, W1.T, b1.reshape(16, 1), W2.T)


def _tc3_call(degp, s2p, g2, b2):
    return pl.pallas_call(
        _tc3_body,
        grid=(_GRID,),
        in_specs=[
            pl.BlockSpec((2, _BL), lambda b: (0, b)),
            pl.BlockSpec((2, _BL), lambda b: (0, b)),
            pl.BlockSpec((1, _BL), lambda b: (0, b)),
            pl.BlockSpec((1, 1), lambda b: (0, 0)),
        ],
        out_specs=pl.BlockSpec((1, _BL), lambda b: (0, b)),
        out_shape=jax.ShapeDtypeStruct((1, NP), jnp.float32),
    )(degp, s2p, g2, b2.reshape(1, 1))


def kernel(x, edge_index, W1, b1, W2, b2):
    src = edge_index[0].astype(jnp.int32)
    dst = edge_index[1].astype(jnp.int32)
    src_p = src.reshape(DESC, LW)
    dst_p = dst.reshape(DESC, LW)
    src_pb = src.reshape(DESCB, LWB)
    dst_pb = dst.reshape(DESCB, LWB)

    x_t = jnp.pad(x.T, ((0, 0), (0, NP - N)))
    zeros_n = jnp.zeros((NP,), jnp.float32)
    zeros_n8 = jnp.zeros((NP, 8), jnp.float32)

    degp = _sc_deg(dst_p, zeros_n)
    xst = _tc1_call(degp, x_t)
    xs = xst.T.reshape(NP, 8)
    a1p = _sc_agg4(xs, src_pb, dst_pb, zeros_n8)
    a1t = a1p.transpose(0, 2, 1)
    g2 = _tc2_call(degp, a1t, x_t, W1, b1, W2)
    s2p = _sc_agg1(g2.reshape(NP), src_p, dst_p, zeros_n)
    out = _tc3_call(degp, s2p, g2, b2)
    return out[0, :N].reshape(N, 1)
